# baseline (speedup 1.0000x reference)
import jax
import jax.numpy as jnp
from jax.experimental import pallas as pl


def _norm_rows(x):
    n = jnp.sqrt(jnp.sum(x * x, axis=1))
    return x / n[:, None]


def kernel(next_feature, adj_new_vals, feature, adj_old_vals, alpha, beta, gamma, persona, edge_new_idx, edge_old_idx, time):
    pt = persona[time]
    d = feature.shape[1]
    n = feature.shape[0]
    src = edge_new_idx[0]
    dst = edge_new_idx[1]
    diff = feature - next_feature
    sim_impact = jnp.sum(diff[src] * diff[dst], axis=1) * adj_new_vals
    impact_norm = sim_impact / d
    persona_gamma = (pt @ gamma.reshape(-1, 1))[:, 0]
    reward_impact = impact_norm * persona_gamma[src]
    normed = _norm_rows(_norm_rows(next_feature))
    sim_vals = jnp.sum(normed[src] * normed[dst], axis=1) * adj_new_vals
    persona_alpha = (pt @ alpha.reshape(-1, 1))[:, 0]
    reward_sim = sim_vals * persona_alpha[src]
    persona_beta = (pt @ beta.reshape(-1, 1))[:, 0]
    so = edge_old_idx[0]
    do = edge_old_idx[1]
    reward_cost = adj_old_vals * persona_beta[so]
    reward = jnp.zeros((n, n), dtype=feature.dtype)
    reward = reward.at[src, dst].add(reward_sim + reward_impact)
    reward = reward.at[so, do].add(-reward_cost)
    return reward


# speedup vs baseline: 4.0206x; 4.0206x over previous
import functools

import jax
import jax.numpy as jnp
from jax import lax
from jax.experimental import pallas as pl
from jax.experimental.pallas import tpu as pltpu
from jax.experimental.pallas import tpu_sc as plsc

_L = 16
_CHUNK = 128
_RROWS = 256
_DUMPW = 1024


def _prep_body(nf_ref, f_ref, ptp_ref, w_ref, a_ref, b_ref, pb_ref):
    nf = nf_ref[...]
    f = f_ref[...]
    d = nf.shape[1]
    rv = jnp.dot(ptp_ref[...], w_ref[...], preferred_element_type=jnp.float32)
    pa = rv[:, 0:1]
    pg = rv[:, 1:2]
    pb = rv[:, 2:3]
    n1 = nf / jnp.sqrt(jnp.sum(nf * nf, axis=1, keepdims=True))
    n2 = n1 / jnp.sqrt(jnp.sum(n1 * n1, axis=1, keepdims=True))
    diff = f - nf
    a_ref[...] = jnp.concatenate([pa * n2, (pg * (1.0 / d)) * diff], axis=1)
    b_ref[...] = jnp.concatenate([n2, diff], axis=1)
    pb_ref[...] = pb.reshape(pb_ref.shape)


def _make_prep(n, d):
    rb = 1024
    return pl.pallas_call(
        _prep_body,
        grid=(n // rb,),
        in_specs=[
            pl.BlockSpec((rb, d), lambda i: (i, 0)),
            pl.BlockSpec((rb, d), lambda i: (i, 0)),
            pl.BlockSpec((rb, d), lambda i: (i, 0)),
            pl.BlockSpec((d, d), lambda i: (0, 0)),
        ],
        out_specs=[
            pl.BlockSpec((rb, 2 * d), lambda i: (i, 0)),
            pl.BlockSpec((rb, 2 * d), lambda i: (i, 0)),
            pl.BlockSpec((rb // 128, 128), lambda i: (i, 0)),
        ],
        out_shape=[
            jax.ShapeDtypeStruct((n, 2 * d), jnp.float32),
            jax.ShapeDtypeStruct((n, 2 * d), jnp.float32),
            jax.ShapeDtypeStruct((n // 128, 128), jnp.float32),
        ],
    )


def _make_edge_vals(n, d, e):
    info = plsc.get_sparse_core_info()
    nc, ns = info.num_cores, info.num_subcores
    nw = nc * ns
    epw = e // nw
    c = _CHUNK
    w2 = 2 * d
    mesh = plsc.VectorSubcoreMesh(core_axis_name="c", subcore_axis_name="s")

    @functools.partial(
        pl.kernel,
        mesh=mesh,
        out_type=(
            jax.ShapeDtypeStruct((e,), jnp.int32),
            jax.ShapeDtypeStruct((e,), jnp.float32),
            jax.ShapeDtypeStruct((e,), jnp.int32),
            jax.ShapeDtypeStruct((e,), jnp.float32),
        ),
        scratch_types=[
            pltpu.VMEM((epw,), jnp.int32),
            pltpu.VMEM((epw,), jnp.int32),
            pltpu.VMEM((epw,), jnp.float32),
            pltpu.VMEM((c, w2), jnp.float32),
            pltpu.VMEM((c, w2), jnp.float32),
            pltpu.VMEM((epw,), jnp.int32),
            pltpu.VMEM((epw,), jnp.float32),
            pltpu.VMEM((n // 128, 128), jnp.float32),
            pltpu.SemaphoreType.DMA,
            pltpu.SemaphoreType.DMA,
        ],
        compiler_params=pltpu.CompilerParams(use_tc_tiling_on_sc=False, needs_layout_passes=False),
    )
    def edge_vals(a_hbm, b_hbm, src_hbm, dst_hbm, adjn_hbm, so_hbm, do_hbm,
                  adjo_hbm, pb_hbm, kn_out, vn_out, ko_out, vo_out,
                  src_v, dst_v, adj_v, rows_a, rows_b, keys_v, vals_v, pb_v,
                  sem_a, sem_b):
        wid = lax.axis_index("s") * nc + lax.axis_index("c")
        base = wid * epw
        lanes = lax.iota(jnp.int32, _L)

        pltpu.sync_copy(src_hbm.at[pl.ds(base, epw)], src_v)
        pltpu.sync_copy(dst_hbm.at[pl.ds(base, epw)], dst_v)
        pltpu.sync_copy(adjn_hbm.at[pl.ds(base, epw)], adj_v)

        def chunk_body(ci, _):
            ca = pltpu.async_copy(a_hbm.at[src_v.at[pl.ds(ci * c, c)]],
                                  rows_a, sem_a)
            cb = pltpu.async_copy(b_hbm.at[dst_v.at[pl.ds(ci * c, c)]],
                                  rows_b, sem_b)
            ca.wait()
            cb.wait()

            def grp(gi, _):
                row = lanes + gi * _L

                def kstep(kk, acc):
                    for u in range(8):
                        col = jnp.full((_L,), kk * 8 + u, jnp.int32)
                        av = plsc.load_gather(rows_a, [row, col])
                        bv = plsc.load_gather(rows_b, [row, col])
                        acc = acc + av * bv
                    return acc

                acc = lax.fori_loop(0, w2 // 8, kstep,
                                    jnp.zeros((_L,), jnp.float32))
                eb = ci * c + gi * _L
                s_g = src_v[pl.ds(eb, _L)]
                d_g = dst_v[pl.ds(eb, _L)]
                a_g = adj_v[pl.ds(eb, _L)]
                keys_v[pl.ds(eb, _L)] = s_g * n + d_g
                vals_v[pl.ds(eb, _L)] = acc * a_g
                return 0

            lax.fori_loop(0, c // _L, grp, 0)
            return 0

        lax.fori_loop(0, epw // c, chunk_body, 0)
        pltpu.sync_copy(keys_v, kn_out.at[pl.ds(base, epw)])
        pltpu.sync_copy(vals_v, vn_out.at[pl.ds(base, epw)])

        pltpu.sync_copy(pb_hbm, pb_v)
        pltpu.sync_copy(so_hbm.at[pl.ds(base, epw)], src_v)
        pltpu.sync_copy(do_hbm.at[pl.ds(base, epw)], dst_v)
        pltpu.sync_copy(adjo_hbm.at[pl.ds(base, epw)], adj_v)

        def ogrp(gi, _):
            eb = gi * _L
            s_g = src_v[pl.ds(eb, _L)]
            d_g = dst_v[pl.ds(eb, _L)]
            a_g = adj_v[pl.ds(eb, _L)]
            pbg = plsc.load_gather(
                pb_v, [jnp.right_shift(s_g, 7), jnp.bitwise_and(s_g, 127)])
            keys_v[pl.ds(eb, _L)] = s_g * n + d_g
            vals_v[pl.ds(eb, _L)] = -(a_g * pbg)
            return 0

        lax.fori_loop(0, epw // _L, ogrp, 0)
        pltpu.sync_copy(keys_v, ko_out.at[pl.ds(base, epw)])
        pltpu.sync_copy(vals_v, vo_out.at[pl.ds(base, epw)])

    return edge_vals


def _make_scatter(n, e):
    info = plsc.get_sparse_core_info()
    nc, ns = info.num_cores, info.num_subcores
    rwords = _RROWS * n
    nranges = (n * n) // rwords
    npass = nranges // nc
    sl = e // ns
    sr = sl // 128
    span = rwords // ns
    mesh = plsc.VectorSubcoreMesh(core_axis_name="c", subcore_axis_name="s")

    @functools.partial(
        pl.kernel,
        mesh=mesh,
        out_type=jax.ShapeDtypeStruct((n * n,), jnp.float32),
        scratch_types=[
            pltpu.VMEM((sr, 128), jnp.int32),
            pltpu.VMEM((sr, 128), jnp.float32),
            pltpu.VMEM((sr, 128), jnp.int32),
            pltpu.VMEM((sr, 128), jnp.float32),
            pltpu.VMEM((sr, 128), jnp.int32),
            pltpu.VMEM((sr, 128), jnp.int32),
            pltpu.VMEM((4096,), jnp.float32),
            pltpu.VMEM_SHARED((rwords + _DUMPW,), jnp.float32),
            pltpu.SemaphoreType.DMA,
        ],
        compiler_params=pltpu.CompilerParams(use_tc_tiling_on_sc=False, needs_layout_passes=False),
    )
    def scatter(kn_hbm, vn_hbm, ko_hbm, vo_hbm, out_hbm,
                knew, vnew, kold, vold, idxn, idxo, zbuf, shared, sem_s):
        cc = lax.axis_index("c")
        s = lax.axis_index("s")

        pltpu.sync_copy(kn_hbm.at[pl.ds(s * sr, sr)], knew)
        pltpu.sync_copy(vn_hbm.at[pl.ds(s * sr, sr)], vnew)
        pltpu.sync_copy(ko_hbm.at[pl.ds(s * sr, sr)], kold)
        pltpu.sync_copy(vo_hbm.at[pl.ds(s * sr, sr)], vold)

        zv = jnp.zeros((_L,), jnp.float32)

        def zb(i, _):
            zbuf[pl.ds(i * _L, _L)] = zv
            return 0

        lax.fori_loop(0, 4096 // _L, zb, 0)

        def pass_body(p, _):
            rid = p * nc + cc
            lo = rid * rwords

            def zr(z, _):
                pltpu.sync_copy(zbuf,
                                shared.at[pl.ds(s * span + z * 4096, 4096)])
                return 0

            lax.fori_loop(0, span // 4096, zr, 0)
            plsc.subcore_barrier()

            def bi(j, _):
                for g in range(8):
                    kk = knew[j, pl.ds(g * _L, _L)]
                    ln = kk - lo
                    inr = (kk >= lo) & (ln < rwords)
                    idxn[j, pl.ds(g * _L, _L)] = jnp.where(
                        inr, ln, rwords + jnp.bitwise_and(kk, _DUMPW - 1))
                    ko = kold[j, pl.ds(g * _L, _L)]
                    lno = ko - lo
                    inro = (ko >= lo) & (lno < rwords)
                    idxo[j, pl.ds(g * _L, _L)] = jnp.where(
                        inro, lno, rwords + jnp.bitwise_and(ko, _DUMPW - 1))
                return 0

            lax.fori_loop(0, sr, bi, 0)

            def fire(j, _):
                pltpu.sync_copy(vnew.at[j], shared.at[idxn.at[j]], add=True)
                pltpu.sync_copy(vold.at[j], shared.at[idxo.at[j]], add=True)
                return 0

            lax.fori_loop(0, sr, fire, 0)
            plsc.subcore_barrier()

            pltpu.sync_copy(shared.at[pl.ds(s * span, span)],
                            out_hbm.at[pl.ds(lo + s * span, span)])
            return 0

        lax.fori_loop(0, npass, pass_body, 0)

    return scatter


def kernel(next_feature, adj_new_vals, feature, adj_old_vals, alpha, beta,
           gamma, persona, edge_new_idx, edge_old_idx, time):
    n, d = feature.shape
    e = edge_new_idx.shape[1]
    p = persona.shape[2]

    pt = lax.dynamic_index_in_dim(persona, time, 0, keepdims=False)
    ptp = jnp.pad(pt, ((0, 0), (0, d - p)))
    w = (jnp.zeros((d, d), jnp.float32)
         .at[:p, 0].set(alpha)
         .at[:p, 1].set(gamma)
         .at[:p, 2].set(beta))

    a, b, pb2 = _make_prep(n, d)(next_feature, feature, ptp, w)

    src = edge_new_idx[0].astype(jnp.int32)
    dst = edge_new_idx[1].astype(jnp.int32)
    so = edge_old_idx[0].astype(jnp.int32)
    do = edge_old_idx[1].astype(jnp.int32)

    kn, vn, ko, vo = _make_edge_vals(n, d, e)(
        a, b, src, dst, adj_new_vals, so, do, adj_old_vals, pb2)

    out_flat = _make_scatter(n, e)(
        kn.reshape(-1, 128), vn.reshape(-1, 128),
        ko.reshape(-1, 128), vo.reshape(-1, 128))

    return out_flat.reshape(n, n)


# speedup vs baseline: 4.2940x; 1.0680x over previous
import functools

import jax
import jax.numpy as jnp
from jax import lax
from jax.experimental import pallas as pl
from jax.experimental.pallas import tpu as pltpu
from jax.experimental.pallas import tpu_sc as plsc

_L = 16
_CHUNK = 128
_RROWS = 256
_DUMPW = 1024


def _prep_body(nf_ref, f_ref, ptp_ref, w_ref, a_ref, b_ref, pb_ref):
    nf = nf_ref[...]
    f = f_ref[...]
    d = nf.shape[1]
    rv = jnp.dot(ptp_ref[...], w_ref[...], preferred_element_type=jnp.float32)
    pa = rv[:, 0:1]
    pg = rv[:, 1:2]
    pb = rv[:, 2:3]
    n1 = nf / jnp.sqrt(jnp.sum(nf * nf, axis=1, keepdims=True))
    n2 = n1 / jnp.sqrt(jnp.sum(n1 * n1, axis=1, keepdims=True))
    diff = f - nf
    a_ref[...] = jnp.concatenate([pa * n2, (pg * (1.0 / d)) * diff], axis=1)
    b_ref[...] = jnp.concatenate([n2, diff], axis=1)
    pb_ref[...] = pb.reshape(pb_ref.shape)


def _make_prep(n, d):
    rb = 1024
    return pl.pallas_call(
        _prep_body,
        grid=(n // rb,),
        in_specs=[
            pl.BlockSpec((rb, d), lambda i: (i, 0)),
            pl.BlockSpec((rb, d), lambda i: (i, 0)),
            pl.BlockSpec((rb, d), lambda i: (i, 0)),
            pl.BlockSpec((d, d), lambda i: (0, 0)),
        ],
        out_specs=[
            pl.BlockSpec((rb, 2 * d), lambda i: (i, 0)),
            pl.BlockSpec((rb, 2 * d), lambda i: (i, 0)),
            pl.BlockSpec((rb // 128, 128), lambda i: (i, 0)),
        ],
        out_shape=[
            jax.ShapeDtypeStruct((n, 2 * d), jnp.float32),
            jax.ShapeDtypeStruct((n, 2 * d), jnp.float32),
            jax.ShapeDtypeStruct((n // 128, 128), jnp.float32),
        ],
    )


def _make_edge_vals(n, d, e):
    info = plsc.get_sparse_core_info()
    nc, ns = info.num_cores, info.num_subcores
    nw = nc * ns
    epw = e // nw
    c = _CHUNK
    w2 = 2 * d
    mesh = plsc.VectorSubcoreMesh(core_axis_name="c", subcore_axis_name="s")

    @functools.partial(
        pl.kernel,
        mesh=mesh,
        out_type=(
            jax.ShapeDtypeStruct((e,), jnp.int32),
            jax.ShapeDtypeStruct((e,), jnp.float32),
            jax.ShapeDtypeStruct((e,), jnp.int32),
            jax.ShapeDtypeStruct((e,), jnp.float32),
        ),
        scratch_types=[
            pltpu.VMEM((epw,), jnp.int32),
            pltpu.VMEM((epw,), jnp.int32),
            pltpu.VMEM((epw,), jnp.float32),
            pltpu.VMEM((c, w2), jnp.float32),
            pltpu.VMEM((c, w2), jnp.float32),
            pltpu.VMEM((epw,), jnp.int32),
            pltpu.VMEM((epw,), jnp.float32),
            pltpu.VMEM((n // 128, 128), jnp.float32),
            pltpu.SemaphoreType.DMA,
            pltpu.SemaphoreType.DMA,
        ],
        compiler_params=pltpu.CompilerParams(use_tc_tiling_on_sc=False, needs_layout_passes=False),
    )
    def edge_vals(a_hbm, b_hbm, src_hbm, dst_hbm, adjn_hbm, so_hbm, do_hbm,
                  adjo_hbm, pb_hbm, kn_out, vn_out, ko_out, vo_out,
                  src_v, dst_v, adj_v, rows_a, rows_b, keys_v, vals_v, pb_v,
                  sem_a, sem_b):
        wid = lax.axis_index("s") * nc + lax.axis_index("c")
        base = wid * epw
        lanes = lax.iota(jnp.int32, _L)

        pltpu.sync_copy(src_hbm.at[pl.ds(base, epw)], src_v)
        pltpu.sync_copy(dst_hbm.at[pl.ds(base, epw)], dst_v)
        pltpu.sync_copy(adjn_hbm.at[pl.ds(base, epw)], adj_v)

        def chunk_body(ci, _):
            ca = pltpu.async_copy(a_hbm.at[src_v.at[pl.ds(ci * c, c)]],
                                  rows_a, sem_a)
            cb = pltpu.async_copy(b_hbm.at[dst_v.at[pl.ds(ci * c, c)]],
                                  rows_b, sem_b)
            ca.wait()
            cb.wait()

            def grp(gi, _):
                row = lanes + gi * _L

                @plsc.parallel_loop(0, w2 // 8, carry=jnp.zeros((_L,),
                                                                jnp.float32),
                                    unroll=2)
                def kacc(kk, acc):
                    ps = []
                    for u in range(8):
                        col = jnp.full((_L,), kk * 8 + u, jnp.int32)
                        av = plsc.load_gather(rows_a, [row, col])
                        bv = plsc.load_gather(rows_b, [row, col])
                        ps.append(av * bv)
                    s = (((ps[0] + ps[1]) + (ps[2] + ps[3]))
                         + ((ps[4] + ps[5]) + (ps[6] + ps[7])))
                    return acc + s

                acc = kacc
                eb = ci * c + gi * _L
                s_g = src_v[pl.ds(eb, _L)]
                d_g = dst_v[pl.ds(eb, _L)]
                a_g = adj_v[pl.ds(eb, _L)]
                keys_v[pl.ds(eb, _L)] = s_g * n + d_g
                vals_v[pl.ds(eb, _L)] = acc * a_g
                return 0

            lax.fori_loop(0, c // _L, grp, 0)
            return 0

        lax.fori_loop(0, epw // c, chunk_body, 0)
        pltpu.sync_copy(keys_v, kn_out.at[pl.ds(base, epw)])
        pltpu.sync_copy(vals_v, vn_out.at[pl.ds(base, epw)])

        pltpu.sync_copy(pb_hbm, pb_v)
        pltpu.sync_copy(so_hbm.at[pl.ds(base, epw)], src_v)
        pltpu.sync_copy(do_hbm.at[pl.ds(base, epw)], dst_v)
        pltpu.sync_copy(adjo_hbm.at[pl.ds(base, epw)], adj_v)

        def ogrp(gi, _):
            eb = gi * _L
            s_g = src_v[pl.ds(eb, _L)]
            d_g = dst_v[pl.ds(eb, _L)]
            a_g = adj_v[pl.ds(eb, _L)]
            pbg = plsc.load_gather(
                pb_v, [jnp.right_shift(s_g, 7), jnp.bitwise_and(s_g, 127)])
            keys_v[pl.ds(eb, _L)] = s_g * n + d_g
            vals_v[pl.ds(eb, _L)] = -(a_g * pbg)
            return 0

        lax.fori_loop(0, epw // _L, ogrp, 0)
        pltpu.sync_copy(keys_v, ko_out.at[pl.ds(base, epw)])
        pltpu.sync_copy(vals_v, vo_out.at[pl.ds(base, epw)])

    return edge_vals


def _make_scatter(n, e):
    info = plsc.get_sparse_core_info()
    nc, ns = info.num_cores, info.num_subcores
    rwords = _RROWS * n
    nranges = (n * n) // rwords
    npass = nranges // nc
    sl = e // ns
    sr = sl // 128
    span = rwords // ns
    mesh = plsc.VectorSubcoreMesh(core_axis_name="c", subcore_axis_name="s")

    @functools.partial(
        pl.kernel,
        mesh=mesh,
        out_type=jax.ShapeDtypeStruct((n * n,), jnp.float32),
        scratch_types=[
            pltpu.VMEM((sr, 128), jnp.int32),
            pltpu.VMEM((sr, 128), jnp.float32),
            pltpu.VMEM((sr, 128), jnp.int32),
            pltpu.VMEM((sr, 128), jnp.float32),
            pltpu.VMEM((sr, 128), jnp.int32),
            pltpu.VMEM((sr, 128), jnp.int32),
            pltpu.VMEM((4096,), jnp.float32),
            pltpu.VMEM_SHARED((rwords + _DUMPW,), jnp.float32),
            pltpu.SemaphoreType.DMA,
        ],
        compiler_params=pltpu.CompilerParams(use_tc_tiling_on_sc=False, needs_layout_passes=False),
    )
    def scatter(kn_hbm, vn_hbm, ko_hbm, vo_hbm, out_hbm,
                knew, vnew, kold, vold, idxn, idxo, zbuf, shared, sem_s):
        cc = lax.axis_index("c")
        s = lax.axis_index("s")

        pltpu.sync_copy(kn_hbm.at[pl.ds(s * sr, sr)], knew)
        pltpu.sync_copy(vn_hbm.at[pl.ds(s * sr, sr)], vnew)
        pltpu.sync_copy(ko_hbm.at[pl.ds(s * sr, sr)], kold)
        pltpu.sync_copy(vo_hbm.at[pl.ds(s * sr, sr)], vold)

        zv = jnp.zeros((_L,), jnp.float32)

        def zb(i, _):
            zbuf[pl.ds(i * _L, _L)] = zv
            return 0

        lax.fori_loop(0, 4096 // _L, zb, 0)

        def pass_body(p, _):
            rid = p * nc + cc
            lo = rid * rwords

            def zr(z, _):
                pltpu.sync_copy(zbuf,
                                shared.at[pl.ds(s * span + z * 4096, 4096)])
                return 0

            lax.fori_loop(0, span // 4096, zr, 0)
            plsc.subcore_barrier()

            def bi(j, _):
                for g in range(8):
                    kk = knew[j, pl.ds(g * _L, _L)]
                    ln = kk - lo
                    inr = (kk >= lo) & (ln < rwords)
                    idxn[j, pl.ds(g * _L, _L)] = jnp.where(
                        inr, ln, rwords + jnp.bitwise_and(kk, _DUMPW - 1))
                    ko = kold[j, pl.ds(g * _L, _L)]
                    lno = ko - lo
                    inro = (ko >= lo) & (lno < rwords)
                    idxo[j, pl.ds(g * _L, _L)] = jnp.where(
                        inro, lno, rwords + jnp.bitwise_and(ko, _DUMPW - 1))
                return 0

            lax.fori_loop(0, sr, bi, 0)

            def fire(j, _):
                pltpu.sync_copy(vnew.at[j], shared.at[idxn.at[j]], add=True)
                pltpu.sync_copy(vold.at[j], shared.at[idxo.at[j]], add=True)
                return 0

            lax.fori_loop(0, sr, fire, 0)
            plsc.subcore_barrier()

            pltpu.sync_copy(shared.at[pl.ds(s * span, span)],
                            out_hbm.at[pl.ds(lo + s * span, span)])
            return 0

        lax.fori_loop(0, npass, pass_body, 0)

    return scatter


def kernel(next_feature, adj_new_vals, feature, adj_old_vals, alpha, beta,
           gamma, persona, edge_new_idx, edge_old_idx, time):
    n, d = feature.shape
    e = edge_new_idx.shape[1]
    p = persona.shape[2]

    pt = lax.dynamic_index_in_dim(persona, time, 0, keepdims=False)
    ptp = jnp.pad(pt, ((0, 0), (0, d - p)))
    w = (jnp.zeros((d, d), jnp.float32)
         .at[:p, 0].set(alpha)
         .at[:p, 1].set(gamma)
         .at[:p, 2].set(beta))

    a, b, pb2 = _make_prep(n, d)(next_feature, feature, ptp, w)

    src = edge_new_idx[0].astype(jnp.int32)
    dst = edge_new_idx[1].astype(jnp.int32)
    so = edge_old_idx[0].astype(jnp.int32)
    do = edge_old_idx[1].astype(jnp.int32)

    kn, vn, ko, vo = _make_edge_vals(n, d, e)(
        a, b, src, dst, adj_new_vals, so, do, adj_old_vals, pb2)

    out_flat = _make_scatter(n, e)(
        kn.reshape(-1, 128), vn.reshape(-1, 128),
        ko.reshape(-1, 128), vo.reshape(-1, 128))

    return out_flat.reshape(n, n)


# speedup vs baseline: 10.9427x; 2.5483x over previous
import functools

import jax
import jax.numpy as jnp
from jax import lax
from jax.experimental import pallas as pl
from jax.experimental.pallas import tpu as pltpu
from jax.experimental.pallas import tpu_sc as plsc

_L = 16
_CHUNK = 128
_RROWS = 256
_DUMPW = 1024


def _prep_body(nf_ref, f_ref, ptp_ref, w_ref, a_ref, b_ref, pb_ref):
    nf = nf_ref[...]
    f = f_ref[...]
    d = nf.shape[1]
    rv = jnp.dot(ptp_ref[...], w_ref[...], preferred_element_type=jnp.float32)
    pa = rv[:, 0:1]
    pg = rv[:, 1:2]
    pb = rv[:, 2:3]
    n1 = nf / jnp.sqrt(jnp.sum(nf * nf, axis=1, keepdims=True))
    n2 = n1 / jnp.sqrt(jnp.sum(n1 * n1, axis=1, keepdims=True))
    diff = f - nf
    a_ref[...] = jnp.concatenate([pa * n2, (pg * (1.0 / d)) * diff], axis=1)
    b_ref[...] = jnp.concatenate([n2, diff], axis=1)
    pb_ref[...] = pb.reshape(pb_ref.shape)


def _make_prep(n, d):
    rb = 1024
    return pl.pallas_call(
        _prep_body,
        grid=(n // rb,),
        in_specs=[
            pl.BlockSpec((rb, d), lambda i: (i, 0)),
            pl.BlockSpec((rb, d), lambda i: (i, 0)),
            pl.BlockSpec((rb, d), lambda i: (i, 0)),
            pl.BlockSpec((d, d), lambda i: (0, 0)),
        ],
        out_specs=[
            pl.BlockSpec((rb, 2 * d), lambda i: (i, 0)),
            pl.BlockSpec((rb, 2 * d), lambda i: (i, 0)),
            pl.BlockSpec((rb // 128, 128), lambda i: (i, 0)),
        ],
        out_shape=[
            jax.ShapeDtypeStruct((n, 2 * d), jnp.float32),
            jax.ShapeDtypeStruct((n, 2 * d), jnp.float32),
            jax.ShapeDtypeStruct((n // 128, 128), jnp.float32),
        ],
    )


def _make_edge_vals(n, d, e):
    info = plsc.get_sparse_core_info()
    nc, ns = info.num_cores, info.num_subcores
    nw = nc * ns
    epw = e // nw
    c = _CHUNK
    w2 = 2 * d
    mesh = plsc.VectorSubcoreMesh(core_axis_name="c", subcore_axis_name="s")

    @functools.partial(
        pl.kernel,
        mesh=mesh,
        out_type=(
            jax.ShapeDtypeStruct((e,), jnp.int32),
            jax.ShapeDtypeStruct((e,), jnp.float32),
            jax.ShapeDtypeStruct((e,), jnp.int32),
            jax.ShapeDtypeStruct((e,), jnp.float32),
        ),
        scratch_types=[
            pltpu.VMEM((epw,), jnp.int32),
            pltpu.VMEM((epw,), jnp.int32),
            pltpu.VMEM((epw,), jnp.float32),
            pltpu.VMEM((c, w2), jnp.float32),
            pltpu.VMEM((c, w2), jnp.float32),
            pltpu.VMEM((epw,), jnp.int32),
            pltpu.VMEM((epw,), jnp.float32),
            pltpu.VMEM((n // 128, 128), jnp.float32),
            pltpu.SemaphoreType.DMA,
            pltpu.SemaphoreType.DMA,
        ],
        compiler_params=pltpu.CompilerParams(use_tc_tiling_on_sc=False, needs_layout_passes=False),
    )
    def edge_vals(a_hbm, b_hbm, src_hbm, dst_hbm, adjn_hbm, so_hbm, do_hbm,
                  adjo_hbm, pb_hbm, kn_out, vn_out, ko_out, vo_out,
                  src_v, dst_v, adj_v, rows_a, rows_b, keys_v, vals_v, pb_v,
                  sem_a, sem_b):
        wid = lax.axis_index("s") * nc + lax.axis_index("c")
        base = wid * epw
        lanes = lax.iota(jnp.int32, _L)

        pltpu.sync_copy(src_hbm.at[pl.ds(base, epw)], src_v)
        pltpu.sync_copy(dst_hbm.at[pl.ds(base, epw)], dst_v)
        pltpu.sync_copy(adjn_hbm.at[pl.ds(base, epw)], adj_v)

        def chunk_body(ci, _):
            ca = pltpu.async_copy(a_hbm.at[src_v.at[pl.ds(ci * c, c)]],
                                  rows_a, sem_a)
            cb = pltpu.async_copy(b_hbm.at[dst_v.at[pl.ds(ci * c, c)]],
                                  rows_b, sem_b)
            ca.wait()
            cb.wait()

            def grp(gi, _):
                row = lanes + gi * _L

                @plsc.parallel_loop(0, w2 // 8, carry=jnp.zeros((_L,),
                                                                jnp.float32),
                                    unroll=2)
                def kacc(kk, acc):
                    ps = []
                    for u in range(8):
                        col = jnp.bitwise_and(
                            jnp.full((_L,), kk * 8 + u, jnp.int32) + lanes,
                            w2 - 1)
                        av = plsc.load_gather(rows_a, [row, col])
                        bv = plsc.load_gather(rows_b, [row, col])
                        ps.append(av * bv)
                    s = (((ps[0] + ps[1]) + (ps[2] + ps[3]))
                         + ((ps[4] + ps[5]) + (ps[6] + ps[7])))
                    return acc + s

                acc = kacc
                eb = ci * c + gi * _L
                s_g = src_v[pl.ds(eb, _L)]
                d_g = dst_v[pl.ds(eb, _L)]
                a_g = adj_v[pl.ds(eb, _L)]
                keys_v[pl.ds(eb, _L)] = s_g * n + d_g
                vals_v[pl.ds(eb, _L)] = acc * a_g
                return 0

            lax.fori_loop(0, c // _L, grp, 0)
            return 0

        lax.fori_loop(0, epw // c, chunk_body, 0)
        pltpu.sync_copy(keys_v, kn_out.at[pl.ds(base, epw)])
        pltpu.sync_copy(vals_v, vn_out.at[pl.ds(base, epw)])

        pltpu.sync_copy(pb_hbm, pb_v)
        pltpu.sync_copy(so_hbm.at[pl.ds(base, epw)], src_v)
        pltpu.sync_copy(do_hbm.at[pl.ds(base, epw)], dst_v)
        pltpu.sync_copy(adjo_hbm.at[pl.ds(base, epw)], adj_v)

        def ogrp(gi, _):
            eb = gi * _L
            s_g = src_v[pl.ds(eb, _L)]
            d_g = dst_v[pl.ds(eb, _L)]
            a_g = adj_v[pl.ds(eb, _L)]
            pbg = plsc.load_gather(
                pb_v, [jnp.right_shift(s_g, 7), jnp.bitwise_and(s_g, 127)])
            keys_v[pl.ds(eb, _L)] = s_g * n + d_g
            vals_v[pl.ds(eb, _L)] = -(a_g * pbg)
            return 0

        lax.fori_loop(0, epw // _L, ogrp, 0)
        pltpu.sync_copy(keys_v, ko_out.at[pl.ds(base, epw)])
        pltpu.sync_copy(vals_v, vo_out.at[pl.ds(base, epw)])

    return edge_vals


def _make_scatter(n, e):
    info = plsc.get_sparse_core_info()
    nc, ns = info.num_cores, info.num_subcores
    rwords = _RROWS * n
    nranges = (n * n) // rwords
    npass = nranges // nc
    sl = e // ns
    sr = sl // 128
    span = rwords // ns
    mesh = plsc.VectorSubcoreMesh(core_axis_name="c", subcore_axis_name="s")

    @functools.partial(
        pl.kernel,
        mesh=mesh,
        out_type=jax.ShapeDtypeStruct((n * n,), jnp.float32),
        scratch_types=[
            pltpu.VMEM((sr, 128), jnp.int32),
            pltpu.VMEM((sr, 128), jnp.float32),
            pltpu.VMEM((sr, 128), jnp.int32),
            pltpu.VMEM((sr, 128), jnp.float32),
            pltpu.VMEM((sr, 128), jnp.int32),
            pltpu.VMEM((sr, 128), jnp.int32),
            pltpu.VMEM((4096,), jnp.float32),
            pltpu.VMEM_SHARED((rwords + _DUMPW,), jnp.float32),
            pltpu.SemaphoreType.DMA,
        ],
        compiler_params=pltpu.CompilerParams(use_tc_tiling_on_sc=False, needs_layout_passes=False),
    )
    def scatter(kn_hbm, vn_hbm, ko_hbm, vo_hbm, out_hbm,
                knew, vnew, kold, vold, idxn, idxo, zbuf, shared, sem_s):
        cc = lax.axis_index("c")
        s = lax.axis_index("s")

        pltpu.sync_copy(kn_hbm.at[pl.ds(s * sr, sr)], knew)
        pltpu.sync_copy(vn_hbm.at[pl.ds(s * sr, sr)], vnew)
        pltpu.sync_copy(ko_hbm.at[pl.ds(s * sr, sr)], kold)
        pltpu.sync_copy(vo_hbm.at[pl.ds(s * sr, sr)], vold)

        zv = jnp.zeros((_L,), jnp.float32)

        def zb(i, _):
            zbuf[pl.ds(i * _L, _L)] = zv
            return 0

        lax.fori_loop(0, 4096 // _L, zb, 0)

        def pass_body(p, _):
            rid = p * nc + cc
            lo = rid * rwords

            def zr(z, _):
                pltpu.sync_copy(zbuf,
                                shared.at[pl.ds(s * span + z * 4096, 4096)])
                return 0

            lax.fori_loop(0, span // 4096, zr, 0)
            plsc.subcore_barrier()

            def bi(j, _):
                for g in range(8):
                    kk = knew[j, pl.ds(g * _L, _L)]
                    ln = kk - lo
                    inr = (kk >= lo) & (ln < rwords)
                    idxn[j, pl.ds(g * _L, _L)] = jnp.where(
                        inr, ln, rwords + jnp.bitwise_and(kk, _DUMPW - 1))
                    ko = kold[j, pl.ds(g * _L, _L)]
                    lno = ko - lo
                    inro = (ko >= lo) & (lno < rwords)
                    idxo[j, pl.ds(g * _L, _L)] = jnp.where(
                        inro, lno, rwords + jnp.bitwise_and(ko, _DUMPW - 1))
                return 0

            lax.fori_loop(0, sr, bi, 0)

            def fire(j, _):
                pltpu.sync_copy(vnew.at[j], shared.at[idxn.at[j]], add=True)
                pltpu.sync_copy(vold.at[j], shared.at[idxo.at[j]], add=True)
                return 0

            lax.fori_loop(0, sr, fire, 0)
            plsc.subcore_barrier()

            pltpu.sync_copy(shared.at[pl.ds(s * span, span)],
                            out_hbm.at[pl.ds(lo + s * span, span)])
            return 0

        lax.fori_loop(0, npass, pass_body, 0)

    return scatter


def kernel(next_feature, adj_new_vals, feature, adj_old_vals, alpha, beta,
           gamma, persona, edge_new_idx, edge_old_idx, time):
    n, d = feature.shape
    e = edge_new_idx.shape[1]
    p = persona.shape[2]

    pt = lax.dynamic_index_in_dim(persona, time, 0, keepdims=False)
    ptp = jnp.pad(pt, ((0, 0), (0, d - p)))
    w = (jnp.zeros((d, d), jnp.float32)
         .at[:p, 0].set(alpha)
         .at[:p, 1].set(gamma)
         .at[:p, 2].set(beta))

    a, b, pb2 = _make_prep(n, d)(next_feature, feature, ptp, w)

    src = edge_new_idx[0].astype(jnp.int32)
    dst = edge_new_idx[1].astype(jnp.int32)
    so = edge_old_idx[0].astype(jnp.int32)
    do = edge_old_idx[1].astype(jnp.int32)

    kn, vn, ko, vo = _make_edge_vals(n, d, e)(
        a, b, src, dst, adj_new_vals, so, do, adj_old_vals, pb2)

    out_flat = _make_scatter(n, e)(
        kn.reshape(-1, 128), vn.reshape(-1, 128),
        ko.reshape(-1, 128), vo.reshape(-1, 128))

    return out_flat.reshape(n, n)


# speedup vs baseline: 12.6189x; 1.1532x over previous
import functools

import jax
import jax.numpy as jnp
from jax import lax
from jax.experimental import pallas as pl
from jax.experimental.pallas import tpu as pltpu
from jax.experimental.pallas import tpu_sc as plsc

_L = 16
_CHUNK = 128
_RROWS = 256
_DUMPW = 1024


def _prep_body(nf_ref, f_ref, ptp_ref, w_ref, a_ref, b_ref, pb_ref):
    nf = nf_ref[...]
    f = f_ref[...]
    d = nf.shape[1]
    rv = jnp.dot(ptp_ref[...], w_ref[...], preferred_element_type=jnp.float32)
    pa = rv[:, 0:1]
    pg = rv[:, 1:2]
    pb = rv[:, 2:3]
    n1 = nf / jnp.sqrt(jnp.sum(nf * nf, axis=1, keepdims=True))
    n2 = n1 / jnp.sqrt(jnp.sum(n1 * n1, axis=1, keepdims=True))
    diff = f - nf
    a_ref[...] = jnp.concatenate([pa * n2, (pg * (1.0 / d)) * diff], axis=1)
    b_ref[...] = jnp.concatenate([n2, diff], axis=1)
    pb_ref[...] = pb.reshape(pb_ref.shape)


def _make_prep(n, d):
    rb = 1024
    return pl.pallas_call(
        _prep_body,
        grid=(n // rb,),
        in_specs=[
            pl.BlockSpec((rb, d), lambda i: (i, 0)),
            pl.BlockSpec((rb, d), lambda i: (i, 0)),
            pl.BlockSpec((rb, d), lambda i: (i, 0)),
            pl.BlockSpec((d, d), lambda i: (0, 0)),
        ],
        out_specs=[
            pl.BlockSpec((rb, 2 * d), lambda i: (i, 0)),
            pl.BlockSpec((rb, 2 * d), lambda i: (i, 0)),
            pl.BlockSpec((rb // 128, 128), lambda i: (i, 0)),
        ],
        out_shape=[
            jax.ShapeDtypeStruct((n, 2 * d), jnp.float32),
            jax.ShapeDtypeStruct((n, 2 * d), jnp.float32),
            jax.ShapeDtypeStruct((n // 128, 128), jnp.float32),
        ],
    )


def _make_edge_vals(n, d, e):
    info = plsc.get_sparse_core_info()
    nc, ns = info.num_cores, info.num_subcores
    nw = nc * ns
    epw = e // nw
    c = _CHUNK
    w2 = 2 * d
    mesh = plsc.VectorSubcoreMesh(core_axis_name="c", subcore_axis_name="s")

    @functools.partial(
        pl.kernel,
        mesh=mesh,
        out_type=(
            jax.ShapeDtypeStruct((e,), jnp.int32),
            jax.ShapeDtypeStruct((e,), jnp.float32),
            jax.ShapeDtypeStruct((e,), jnp.int32),
            jax.ShapeDtypeStruct((e,), jnp.float32),
        ),
        scratch_types=[
            pltpu.VMEM((epw,), jnp.int32),
            pltpu.VMEM((epw,), jnp.int32),
            pltpu.VMEM((epw,), jnp.float32),
            pltpu.VMEM((c, w2), jnp.float32),
            pltpu.VMEM((c, w2), jnp.float32),
            pltpu.VMEM((epw,), jnp.int32),
            pltpu.VMEM((epw,), jnp.float32),
            pltpu.VMEM((n // 128, 128), jnp.float32),
            pltpu.SemaphoreType.DMA,
            pltpu.SemaphoreType.DMA,
        ],
        compiler_params=pltpu.CompilerParams(use_tc_tiling_on_sc=False, needs_layout_passes=False),
    )
    def edge_vals(a_hbm, b_hbm, src_hbm, dst_hbm, adjn_hbm, so_hbm, do_hbm,
                  adjo_hbm, pb_hbm, kn_out, vn_out, ko_out, vo_out,
                  src_v, dst_v, adj_v, rows_a, rows_b, keys_v, vals_v, pb_v,
                  sem_a, sem_b):
        wid = lax.axis_index("s") * nc + lax.axis_index("c")
        base = wid * epw
        lanes = lax.iota(jnp.int32, _L)

        pltpu.sync_copy(src_hbm.at[pl.ds(base, epw)], src_v)
        pltpu.sync_copy(dst_hbm.at[pl.ds(base, epw)], dst_v)
        pltpu.sync_copy(adjn_hbm.at[pl.ds(base, epw)], adj_v)

        def chunk_body(ci, _):
            ca = pltpu.async_copy(a_hbm.at[src_v.at[pl.ds(ci * c, c)]],
                                  rows_a, sem_a)
            cb = pltpu.async_copy(b_hbm.at[dst_v.at[pl.ds(ci * c, c)]],
                                  rows_b, sem_b)
            ca.wait()
            cb.wait()

            def grp(gi, _):
                row = lanes + gi * _L

                @plsc.parallel_loop(0, w2 // 8, carry=jnp.zeros((_L,),
                                                                jnp.float32),
                                    unroll=2)
                def kacc(kk, acc):
                    ps = []
                    for u in range(8):
                        col = jnp.bitwise_and(
                            jnp.full((_L,), kk * 8 + u, jnp.int32) + lanes,
                            w2 - 1)
                        av = plsc.load_gather(rows_a, [row, col])
                        bv = plsc.load_gather(rows_b, [row, col])
                        ps.append(av * bv)
                    s = (((ps[0] + ps[1]) + (ps[2] + ps[3]))
                         + ((ps[4] + ps[5]) + (ps[6] + ps[7])))
                    return acc + s

                acc = kacc
                eb = ci * c + gi * _L
                s_g = src_v[pl.ds(eb, _L)]
                d_g = dst_v[pl.ds(eb, _L)]
                a_g = adj_v[pl.ds(eb, _L)]
                keys_v[pl.ds(eb, _L)] = s_g * n + d_g
                vals_v[pl.ds(eb, _L)] = acc * a_g
                return 0

            lax.fori_loop(0, c // _L, grp, 0)
            return 0

        lax.fori_loop(0, epw // c, chunk_body, 0)
        pltpu.sync_copy(keys_v, kn_out.at[pl.ds(base, epw)])
        pltpu.sync_copy(vals_v, vn_out.at[pl.ds(base, epw)])

        pltpu.sync_copy(pb_hbm, pb_v)
        pltpu.sync_copy(so_hbm.at[pl.ds(base, epw)], src_v)
        pltpu.sync_copy(do_hbm.at[pl.ds(base, epw)], dst_v)
        pltpu.sync_copy(adjo_hbm.at[pl.ds(base, epw)], adj_v)

        def ogrp(gi, _):
            eb = gi * _L
            s_g = src_v[pl.ds(eb, _L)]
            d_g = dst_v[pl.ds(eb, _L)]
            a_g = adj_v[pl.ds(eb, _L)]
            pbg = plsc.load_gather(
                pb_v, [jnp.right_shift(s_g, 7), jnp.bitwise_and(s_g, 127)])
            keys_v[pl.ds(eb, _L)] = s_g * n + d_g
            vals_v[pl.ds(eb, _L)] = -(a_g * pbg)
            return 0

        lax.fori_loop(0, epw // _L, ogrp, 0)
        pltpu.sync_copy(keys_v, ko_out.at[pl.ds(base, epw)])
        pltpu.sync_copy(vals_v, vo_out.at[pl.ds(base, epw)])

    return edge_vals


def _make_scatter(n, e):
    info = plsc.get_sparse_core_info()
    nc, ns = info.num_cores, info.num_subcores
    rwords = _RROWS * n
    nranges = (n * n) // rwords
    npass = nranges // nc
    sl = e // ns
    sr = sl // 128
    span = rwords // ns
    mesh = plsc.VectorSubcoreMesh(core_axis_name="c", subcore_axis_name="s")

    @functools.partial(
        pl.kernel,
        mesh=mesh,
        out_type=jax.ShapeDtypeStruct((n * n,), jnp.float32),
        scratch_types=[
            pltpu.VMEM((sr, 128), jnp.int32),
            pltpu.VMEM((sr, 128), jnp.float32),
            pltpu.VMEM((sr, 128), jnp.int32),
            pltpu.VMEM((sr, 128), jnp.float32),
            pltpu.VMEM((sr, 128), jnp.int32),
            pltpu.VMEM((sr, 128), jnp.int32),
            pltpu.VMEM((4096,), jnp.float32),
            pltpu.VMEM_SHARED((rwords + _DUMPW,), jnp.float32),
            pltpu.SemaphoreType.DMA,
        ],
        compiler_params=pltpu.CompilerParams(use_tc_tiling_on_sc=False, needs_layout_passes=False),
    )
    def scatter(kn_hbm, vn_hbm, ko_hbm, vo_hbm, out_hbm,
                knew, vnew, kold, vold, idxn, idxo, zbuf, shared, sem_s):
        cc = lax.axis_index("c")
        s = lax.axis_index("s")

        pltpu.sync_copy(kn_hbm.at[pl.ds(s * sr, sr)], knew)
        pltpu.sync_copy(vn_hbm.at[pl.ds(s * sr, sr)], vnew)
        pltpu.sync_copy(ko_hbm.at[pl.ds(s * sr, sr)], kold)
        pltpu.sync_copy(vo_hbm.at[pl.ds(s * sr, sr)], vold)

        zv = jnp.zeros((_L,), jnp.float32)

        def zb(i, _):
            zbuf[pl.ds(i * _L, _L)] = zv
            return 0

        lax.fori_loop(0, 4096 // _L, zb, 0)

        def pass_body(p, _):
            rid = p * nc + cc
            lo = rid * rwords

            def zr(z, _):
                pltpu.async_copy(zbuf,
                                 shared.at[pl.ds(s * span + z * 4096, 4096)],
                                 sem_s)
                return 0

            lax.fori_loop(0, span // 4096, zr, 0)

            def zr_wait(z, _):
                pltpu.make_async_copy(
                    zbuf, shared.at[pl.ds(s * span + z * 4096, 4096)],
                    sem_s).wait()
                return 0

            lax.fori_loop(0, span // 4096, zr_wait, 0)
            plsc.subcore_barrier()

            def bi(j, _):
                for g in range(8):
                    kk = knew[j, pl.ds(g * _L, _L)]
                    ln = kk - lo
                    inr = (kk >= lo) & (ln < rwords)
                    idxn[j, pl.ds(g * _L, _L)] = jnp.where(
                        inr, ln, rwords + jnp.bitwise_and(kk, _DUMPW - 1))
                    ko = kold[j, pl.ds(g * _L, _L)]
                    lno = ko - lo
                    inro = (ko >= lo) & (lno < rwords)
                    idxo[j, pl.ds(g * _L, _L)] = jnp.where(
                        inro, lno, rwords + jnp.bitwise_and(ko, _DUMPW - 1))
                return 0

            lax.fori_loop(0, sr, bi, 0)

            def fire(j, _):
                pltpu.async_copy(vnew.at[j], shared.at[idxn.at[j]], sem_s,
                                 add=True)
                pltpu.async_copy(vold.at[j], shared.at[idxo.at[j]], sem_s,
                                 add=True)
                return 0

            lax.fori_loop(0, sr, fire, 0)

            def drain(j, _):
                pltpu.make_async_copy(vnew.at[j], shared.at[idxn.at[j]],
                                      sem_s).wait()
                pltpu.make_async_copy(vold.at[j], shared.at[idxo.at[j]],
                                      sem_s).wait()
                return 0

            lax.fori_loop(0, sr, drain, 0)
            plsc.subcore_barrier()

            pltpu.sync_copy(shared.at[pl.ds(s * span, span)],
                            out_hbm.at[pl.ds(lo + s * span, span)])
            return 0

        lax.fori_loop(0, npass, pass_body, 0)

    return scatter


def kernel(next_feature, adj_new_vals, feature, adj_old_vals, alpha, beta,
           gamma, persona, edge_new_idx, edge_old_idx, time):
    n, d = feature.shape
    e = edge_new_idx.shape[1]
    p = persona.shape[2]

    pt = lax.dynamic_index_in_dim(persona, time, 0, keepdims=False)
    ptp = jnp.pad(pt, ((0, 0), (0, d - p)))
    w = (jnp.zeros((d, d), jnp.float32)
         .at[:p, 0].set(alpha)
         .at[:p, 1].set(gamma)
         .at[:p, 2].set(beta))

    a, b, pb2 = _make_prep(n, d)(next_feature, feature, ptp, w)

    src = edge_new_idx[0].astype(jnp.int32)
    dst = edge_new_idx[1].astype(jnp.int32)
    so = edge_old_idx[0].astype(jnp.int32)
    do = edge_old_idx[1].astype(jnp.int32)

    kn, vn, ko, vo = _make_edge_vals(n, d, e)(
        a, b, src, dst, adj_new_vals, so, do, adj_old_vals, pb2)

    out_flat = _make_scatter(n, e)(
        kn.reshape(-1, 128), vn.reshape(-1, 128),
        ko.reshape(-1, 128), vo.reshape(-1, 128))

    return out_flat.reshape(n, n)


# speedup vs baseline: 15.2541x; 1.2088x over previous
import functools

import jax
import jax.numpy as jnp
from jax import lax
from jax.experimental import pallas as pl
from jax.experimental.pallas import tpu as pltpu
from jax.experimental.pallas import tpu_sc as plsc

_L = 16
_CHUNK = 64
_RROWS = 256
_DUMPW = 1024


def _prep_body(nf_ref, f_ref, ptp_ref, w_ref, a_ref, b_ref, pb_ref):
    nf = nf_ref[...]
    f = f_ref[...]
    d = nf.shape[1]
    rv = jnp.dot(ptp_ref[...], w_ref[...], preferred_element_type=jnp.float32)
    pa = rv[:, 0:1]
    pg = rv[:, 1:2]
    pb = rv[:, 2:3]
    n1 = nf / jnp.sqrt(jnp.sum(nf * nf, axis=1, keepdims=True))
    n2 = n1 / jnp.sqrt(jnp.sum(n1 * n1, axis=1, keepdims=True))
    diff = f - nf
    a_ref[...] = jnp.concatenate([pa * n2, (pg * (1.0 / d)) * diff], axis=1)
    b_ref[...] = jnp.concatenate([n2, diff], axis=1)
    pb_ref[...] = pb.reshape(pb_ref.shape)


def _make_prep(n, d):
    rb = 1024
    return pl.pallas_call(
        _prep_body,
        grid=(n // rb,),
        in_specs=[
            pl.BlockSpec((rb, d), lambda i: (i, 0)),
            pl.BlockSpec((rb, d), lambda i: (i, 0)),
            pl.BlockSpec((rb, d), lambda i: (i, 0)),
            pl.BlockSpec((d, d), lambda i: (0, 0)),
        ],
        out_specs=[
            pl.BlockSpec((rb, 2 * d), lambda i: (i, 0)),
            pl.BlockSpec((rb, 2 * d), lambda i: (i, 0)),
            pl.BlockSpec((rb // 128, 128), lambda i: (i, 0)),
        ],
        out_shape=[
            jax.ShapeDtypeStruct((n, 2 * d), jnp.float32),
            jax.ShapeDtypeStruct((n, 2 * d), jnp.float32),
            jax.ShapeDtypeStruct((n // 128, 128), jnp.float32),
        ],
    )


def _make_edge_vals(n, d, e):
    info = plsc.get_sparse_core_info()
    nc, ns = info.num_cores, info.num_subcores
    nw = nc * ns
    epw = e // nw
    c = _CHUNK
    w2 = 2 * d
    mesh = plsc.VectorSubcoreMesh(core_axis_name="c", subcore_axis_name="s")

    @functools.partial(
        pl.kernel,
        mesh=mesh,
        out_type=(
            jax.ShapeDtypeStruct((e,), jnp.int32),
            jax.ShapeDtypeStruct((e,), jnp.float32),
            jax.ShapeDtypeStruct((e,), jnp.int32),
            jax.ShapeDtypeStruct((e,), jnp.float32),
        ),
        scratch_types=[
            pltpu.VMEM((epw,), jnp.int32),
            pltpu.VMEM((epw,), jnp.int32),
            pltpu.VMEM((epw,), jnp.float32),
            pltpu.VMEM((c, w2), jnp.float32),
            pltpu.VMEM((c, w2), jnp.float32),
            pltpu.VMEM((c, w2), jnp.float32),
            pltpu.VMEM((c, w2), jnp.float32),
            pltpu.VMEM((epw,), jnp.int32),
            pltpu.VMEM((epw,), jnp.float32),
            pltpu.VMEM((n // 128, 128), jnp.float32),
            pltpu.SemaphoreType.DMA,
            pltpu.SemaphoreType.DMA,
            pltpu.SemaphoreType.DMA,
            pltpu.SemaphoreType.DMA,
        ],
        compiler_params=pltpu.CompilerParams(use_tc_tiling_on_sc=False, needs_layout_passes=False),
    )
    def edge_vals(a_hbm, b_hbm, src_hbm, dst_hbm, adjn_hbm, so_hbm, do_hbm,
                  adjo_hbm, pb_hbm, kn_out, vn_out, ko_out, vo_out,
                  src_v, dst_v, adj_v, rows_a0, rows_b0, rows_a1, rows_b1,
                  keys_v, vals_v, pb_v, sem_a0, sem_b0, sem_a1, sem_b1):
        wid = lax.axis_index("s") * nc + lax.axis_index("c")
        base = wid * epw
        lanes = lax.iota(jnp.int32, _L)

        pltpu.sync_copy(src_hbm.at[pl.ds(base, epw)], src_v)
        pltpu.sync_copy(dst_hbm.at[pl.ds(base, epw)], dst_v)
        pltpu.sync_copy(adjn_hbm.at[pl.ds(base, epw)], adj_v)

        def fire(ci, ra, rb, sa, sb):
            pltpu.async_copy(a_hbm.at[src_v.at[pl.ds(ci * c, c)]], ra, sa)
            pltpu.async_copy(b_hbm.at[dst_v.at[pl.ds(ci * c, c)]], rb, sb)

        def wait(ci, ra, rb, sa, sb):
            pltpu.make_async_copy(a_hbm.at[src_v.at[pl.ds(ci * c, c)]],
                                  ra, sa).wait()
            pltpu.make_async_copy(b_hbm.at[dst_v.at[pl.ds(ci * c, c)]],
                                  rb, sb).wait()

        def compute_chunk(ci, rows_a, rows_b):
            def grp(gi, _):
                row = lanes + gi * _L

                @plsc.parallel_loop(0, w2 // 8, carry=jnp.zeros((_L,),
                                                                jnp.float32),
                                    unroll=2)
                def kacc(kk, acc):
                    ps = []
                    for u in range(8):
                        col = jnp.bitwise_and(
                            jnp.full((_L,), kk * 8 + u, jnp.int32) + lanes,
                            w2 - 1)
                        av = plsc.load_gather(rows_a, [row, col])
                        bv = plsc.load_gather(rows_b, [row, col])
                        ps.append(av * bv)
                    s = (((ps[0] + ps[1]) + (ps[2] + ps[3]))
                         + ((ps[4] + ps[5]) + (ps[6] + ps[7])))
                    return acc + s

                acc = kacc
                eb = ci * c + gi * _L
                s_g = src_v[pl.ds(eb, _L)]
                d_g = dst_v[pl.ds(eb, _L)]
                a_g = adj_v[pl.ds(eb, _L)]
                keys_v[pl.ds(eb, _L)] = s_g * n + d_g
                vals_v[pl.ds(eb, _L)] = acc * a_g
                return 0

            lax.fori_loop(0, c // _L, grp, 0)

        nchunks = epw // c
        fire(0, rows_a0, rows_b0, sem_a0, sem_b0)
        fire(1, rows_a1, rows_b1, sem_a1, sem_b1)

        def chunk_pair(i, _):
            ci0 = i * 2
            wait(ci0, rows_a0, rows_b0, sem_a0, sem_b0)
            compute_chunk(ci0, rows_a0, rows_b0)
            fire(ci0 + 2, rows_a0, rows_b0, sem_a0, sem_b0)
            wait(ci0 + 1, rows_a1, rows_b1, sem_a1, sem_b1)
            compute_chunk(ci0 + 1, rows_a1, rows_b1)
            fire(ci0 + 3, rows_a1, rows_b1, sem_a1, sem_b1)
            return 0

        lax.fori_loop(0, nchunks // 2 - 1, chunk_pair, 0)
        wait(nchunks - 2, rows_a0, rows_b0, sem_a0, sem_b0)
        compute_chunk(nchunks - 2, rows_a0, rows_b0)
        wait(nchunks - 1, rows_a1, rows_b1, sem_a1, sem_b1)
        compute_chunk(nchunks - 1, rows_a1, rows_b1)
        pltpu.sync_copy(keys_v, kn_out.at[pl.ds(base, epw)])
        pltpu.sync_copy(vals_v, vn_out.at[pl.ds(base, epw)])

        pltpu.sync_copy(pb_hbm, pb_v)
        pltpu.sync_copy(so_hbm.at[pl.ds(base, epw)], src_v)
        pltpu.sync_copy(do_hbm.at[pl.ds(base, epw)], dst_v)
        pltpu.sync_copy(adjo_hbm.at[pl.ds(base, epw)], adj_v)

        def ogrp(gi, _):
            eb = gi * _L
            s_g = src_v[pl.ds(eb, _L)]
            d_g = dst_v[pl.ds(eb, _L)]
            a_g = adj_v[pl.ds(eb, _L)]
            pbg = plsc.load_gather(
                pb_v, [jnp.right_shift(s_g, 7), jnp.bitwise_and(s_g, 127)])
            keys_v[pl.ds(eb, _L)] = s_g * n + d_g
            vals_v[pl.ds(eb, _L)] = -(a_g * pbg)
            return 0

        lax.fori_loop(0, epw // _L, ogrp, 0)
        pltpu.sync_copy(keys_v, ko_out.at[pl.ds(base, epw)])
        pltpu.sync_copy(vals_v, vo_out.at[pl.ds(base, epw)])

    return edge_vals


def _make_scatter(n, e):
    info = plsc.get_sparse_core_info()
    nc, ns = info.num_cores, info.num_subcores
    rwords = _RROWS * n
    nranges = (n * n) // rwords
    npass = nranges // nc
    sl = e // ns
    sr = sl // 128
    span = rwords // ns
    mesh = plsc.VectorSubcoreMesh(core_axis_name="c", subcore_axis_name="s")

    @functools.partial(
        pl.kernel,
        mesh=mesh,
        out_type=jax.ShapeDtypeStruct((n * n,), jnp.float32),
        scratch_types=[
            pltpu.VMEM((sr, 128), jnp.int32),
            pltpu.VMEM((sr, 128), jnp.float32),
            pltpu.VMEM((sr, 128), jnp.int32),
            pltpu.VMEM((sr, 128), jnp.float32),
            pltpu.VMEM((sr, 128), jnp.int32),
            pltpu.VMEM((sr, 128), jnp.int32),
            pltpu.VMEM((4096,), jnp.float32),
            pltpu.VMEM_SHARED((rwords + _DUMPW,), jnp.float32),
            pltpu.SemaphoreType.DMA,
        ],
        compiler_params=pltpu.CompilerParams(use_tc_tiling_on_sc=False, needs_layout_passes=False),
    )
    def scatter(kn_hbm, vn_hbm, ko_hbm, vo_hbm, out_hbm,
                knew, vnew, kold, vold, idxn, idxo, zbuf, shared, sem_s):
        cc = lax.axis_index("c")
        s = lax.axis_index("s")

        pltpu.sync_copy(kn_hbm.at[pl.ds(s * sr, sr)], knew)
        pltpu.sync_copy(vn_hbm.at[pl.ds(s * sr, sr)], vnew)
        pltpu.sync_copy(ko_hbm.at[pl.ds(s * sr, sr)], kold)
        pltpu.sync_copy(vo_hbm.at[pl.ds(s * sr, sr)], vold)

        zv = jnp.zeros((_L,), jnp.float32)

        def zb(i, _):
            zbuf[pl.ds(i * _L, _L)] = zv
            return 0

        lax.fori_loop(0, 4096 // _L, zb, 0)

        def pass_body(p, _):
            rid = p * nc + cc
            lo = rid * rwords

            def zr(z, _):
                pltpu.async_copy(zbuf,
                                 shared.at[pl.ds(s * span + z * 4096, 4096)],
                                 sem_s)
                return 0

            lax.fori_loop(0, span // 4096, zr, 0)

            def zr_wait(z, _):
                pltpu.make_async_copy(
                    zbuf, shared.at[pl.ds(s * span + z * 4096, 4096)],
                    sem_s).wait()
                return 0

            lax.fori_loop(0, span // 4096, zr_wait, 0)
            plsc.subcore_barrier()

            def bi(j, _):
                for g in range(8):
                    kk = knew[j, pl.ds(g * _L, _L)]
                    ln = kk - lo
                    inr = (kk >= lo) & (ln < rwords)
                    idxn[j, pl.ds(g * _L, _L)] = jnp.where(
                        inr, ln, rwords + jnp.bitwise_and(kk, _DUMPW - 1))
                    ko = kold[j, pl.ds(g * _L, _L)]
                    lno = ko - lo
                    inro = (ko >= lo) & (lno < rwords)
                    idxo[j, pl.ds(g * _L, _L)] = jnp.where(
                        inro, lno, rwords + jnp.bitwise_and(ko, _DUMPW - 1))
                return 0

            lax.fori_loop(0, sr, bi, 0)

            def fire(j, _):
                pltpu.async_copy(vnew.at[j], shared.at[idxn.at[j]], sem_s,
                                 add=True)
                pltpu.async_copy(vold.at[j], shared.at[idxo.at[j]], sem_s,
                                 add=True)
                return 0

            lax.fori_loop(0, sr, fire, 0)

            def drain(j, _):
                pltpu.make_async_copy(vnew.at[j], shared.at[idxn.at[j]],
                                      sem_s).wait()
                pltpu.make_async_copy(vold.at[j], shared.at[idxo.at[j]],
                                      sem_s).wait()
                return 0

            lax.fori_loop(0, sr, drain, 0)
            plsc.subcore_barrier()

            pltpu.sync_copy(shared.at[pl.ds(s * span, span)],
                            out_hbm.at[pl.ds(lo + s * span, span)])
            return 0

        lax.fori_loop(0, npass, pass_body, 0)

    return scatter


def kernel(next_feature, adj_new_vals, feature, adj_old_vals, alpha, beta,
           gamma, persona, edge_new_idx, edge_old_idx, time):
    n, d = feature.shape
    e = edge_new_idx.shape[1]
    p = persona.shape[2]

    pt = lax.dynamic_index_in_dim(persona, time, 0, keepdims=False)
    ptp = jnp.pad(pt, ((0, 0), (0, d - p)))
    w = (jnp.zeros((d, d), jnp.float32)
         .at[:p, 0].set(alpha)
         .at[:p, 1].set(gamma)
         .at[:p, 2].set(beta))

    a, b, pb2 = _make_prep(n, d)(next_feature, feature, ptp, w)

    src = edge_new_idx[0].astype(jnp.int32)
    dst = edge_new_idx[1].astype(jnp.int32)
    so = edge_old_idx[0].astype(jnp.int32)
    do = edge_old_idx[1].astype(jnp.int32)

    kn, vn, ko, vo = _make_edge_vals(n, d, e)(
        a, b, src, dst, adj_new_vals, so, do, adj_old_vals, pb2)

    out_flat = _make_scatter(n, e)(
        kn.reshape(-1, 128), vn.reshape(-1, 128),
        ko.reshape(-1, 128), vo.reshape(-1, 128))

    return out_flat.reshape(n, n)


# speedup vs baseline: 16.3481x; 1.0717x over previous
import functools

import jax
import jax.numpy as jnp
from jax import lax
from jax.experimental import pallas as pl
from jax.experimental.pallas import tpu as pltpu
from jax.experimental.pallas import tpu_sc as plsc

_L = 16
_CHUNK = 64
_RROWS = 256
_NPASS = 8
_BCAP = 2048


def _prep_body(nf_ref, f_ref, ptp_ref, w_ref, a_ref, b_ref, pb_ref):
    nf = nf_ref[...]
    f = f_ref[...]
    d = nf.shape[1]
    rv = jnp.dot(ptp_ref[...], w_ref[...], preferred_element_type=jnp.float32)
    pa = rv[:, 0:1]
    pg = rv[:, 1:2]
    pb = rv[:, 2:3]
    n1 = nf / jnp.sqrt(jnp.sum(nf * nf, axis=1, keepdims=True))
    n2 = n1 / jnp.sqrt(jnp.sum(n1 * n1, axis=1, keepdims=True))
    diff = f - nf
    a_ref[...] = jnp.concatenate([pa * n2, (pg * (1.0 / d)) * diff], axis=1)
    b_ref[...] = jnp.concatenate([n2, diff], axis=1)
    pb_ref[...] = pb.reshape(pb_ref.shape)


def _make_prep(n, d):
    rb = 1024
    return pl.pallas_call(
        _prep_body,
        grid=(n // rb,),
        in_specs=[
            pl.BlockSpec((rb, d), lambda i: (i, 0)),
            pl.BlockSpec((rb, d), lambda i: (i, 0)),
            pl.BlockSpec((rb, d), lambda i: (i, 0)),
            pl.BlockSpec((d, d), lambda i: (0, 0)),
        ],
        out_specs=[
            pl.BlockSpec((rb, 2 * d), lambda i: (i, 0)),
            pl.BlockSpec((rb, 2 * d), lambda i: (i, 0)),
            pl.BlockSpec((rb // 128, 128), lambda i: (i, 0)),
        ],
        out_shape=[
            jax.ShapeDtypeStruct((n, 2 * d), jnp.float32),
            jax.ShapeDtypeStruct((n, 2 * d), jnp.float32),
            jax.ShapeDtypeStruct((n // 128, 128), jnp.float32),
        ],
    )


def _make_edge_vals(n, d, e):
    info = plsc.get_sparse_core_info()
    nc, ns = info.num_cores, info.num_subcores
    nw = nc * ns
    epw = e // nw
    c = _CHUNK
    w2 = 2 * d
    mesh = plsc.VectorSubcoreMesh(core_axis_name="c", subcore_axis_name="s")

    @functools.partial(
        pl.kernel,
        mesh=mesh,
        out_type=(
            jax.ShapeDtypeStruct((e,), jnp.int32),
            jax.ShapeDtypeStruct((e,), jnp.float32),
            jax.ShapeDtypeStruct((e,), jnp.int32),
            jax.ShapeDtypeStruct((e,), jnp.float32),
        ),
        scratch_types=[
            pltpu.VMEM((epw,), jnp.int32),
            pltpu.VMEM((epw,), jnp.int32),
            pltpu.VMEM((epw,), jnp.float32),
            pltpu.VMEM((c, w2), jnp.float32),
            pltpu.VMEM((c, w2), jnp.float32),
            pltpu.VMEM((c, w2), jnp.float32),
            pltpu.VMEM((c, w2), jnp.float32),
            pltpu.VMEM((epw,), jnp.int32),
            pltpu.VMEM((epw,), jnp.float32),
            pltpu.VMEM((n // 128, 128), jnp.float32),
            pltpu.SemaphoreType.DMA,
            pltpu.SemaphoreType.DMA,
            pltpu.SemaphoreType.DMA,
            pltpu.SemaphoreType.DMA,
        ],
        compiler_params=pltpu.CompilerParams(use_tc_tiling_on_sc=False, needs_layout_passes=False),
    )
    def edge_vals(a_hbm, b_hbm, src_hbm, dst_hbm, adjn_hbm, so_hbm, do_hbm,
                  adjo_hbm, pb_hbm, kn_out, vn_out, ko_out, vo_out,
                  src_v, dst_v, adj_v, rows_a0, rows_b0, rows_a1, rows_b1,
                  keys_v, vals_v, pb_v, sem_a0, sem_b0, sem_a1, sem_b1):
        wid = lax.axis_index("s") * nc + lax.axis_index("c")
        base = wid * epw
        lanes = lax.iota(jnp.int32, _L)

        pltpu.sync_copy(src_hbm.at[pl.ds(base, epw)], src_v)
        pltpu.sync_copy(dst_hbm.at[pl.ds(base, epw)], dst_v)
        pltpu.sync_copy(adjn_hbm.at[pl.ds(base, epw)], adj_v)

        def fire(ci, ra, rb, sa, sb):
            pltpu.async_copy(a_hbm.at[src_v.at[pl.ds(ci * c, c)]], ra, sa)
            pltpu.async_copy(b_hbm.at[dst_v.at[pl.ds(ci * c, c)]], rb, sb)

        def wait(ci, ra, rb, sa, sb):
            pltpu.make_async_copy(a_hbm.at[src_v.at[pl.ds(ci * c, c)]],
                                  ra, sa).wait()
            pltpu.make_async_copy(b_hbm.at[dst_v.at[pl.ds(ci * c, c)]],
                                  rb, sb).wait()

        def compute_chunk(ci, rows_a, rows_b):
            def grp(gi, _):
                row = lanes + gi * _L

                @plsc.parallel_loop(0, w2 // 8, carry=jnp.zeros((_L,),
                                                                jnp.float32),
                                    unroll=2)
                def kacc(kk, acc):
                    ps = []
                    for u in range(8):
                        col = jnp.bitwise_and(
                            jnp.full((_L,), kk * 8 + u, jnp.int32) + lanes,
                            w2 - 1)
                        av = plsc.load_gather(rows_a, [row, col])
                        bv = plsc.load_gather(rows_b, [row, col])
                        ps.append(av * bv)
                    s = (((ps[0] + ps[1]) + (ps[2] + ps[3]))
                         + ((ps[4] + ps[5]) + (ps[6] + ps[7])))
                    return acc + s

                acc = kacc
                eb = ci * c + gi * _L
                s_g = src_v[pl.ds(eb, _L)]
                d_g = dst_v[pl.ds(eb, _L)]
                a_g = adj_v[pl.ds(eb, _L)]
                keys_v[pl.ds(eb, _L)] = s_g * n + d_g
                vals_v[pl.ds(eb, _L)] = acc * a_g
                return 0

            lax.fori_loop(0, c // _L, grp, 0)

        nchunks = epw // c
        fire(0, rows_a0, rows_b0, sem_a0, sem_b0)
        fire(1, rows_a1, rows_b1, sem_a1, sem_b1)

        def chunk_pair(i, _):
            ci0 = i * 2
            wait(ci0, rows_a0, rows_b0, sem_a0, sem_b0)
            compute_chunk(ci0, rows_a0, rows_b0)
            fire(ci0 + 2, rows_a0, rows_b0, sem_a0, sem_b0)
            wait(ci0 + 1, rows_a1, rows_b1, sem_a1, sem_b1)
            compute_chunk(ci0 + 1, rows_a1, rows_b1)
            fire(ci0 + 3, rows_a1, rows_b1, sem_a1, sem_b1)
            return 0

        lax.fori_loop(0, nchunks // 2 - 1, chunk_pair, 0)
        wait(nchunks - 2, rows_a0, rows_b0, sem_a0, sem_b0)
        compute_chunk(nchunks - 2, rows_a0, rows_b0)
        wait(nchunks - 1, rows_a1, rows_b1, sem_a1, sem_b1)
        compute_chunk(nchunks - 1, rows_a1, rows_b1)
        pltpu.sync_copy(keys_v, kn_out.at[pl.ds(base, epw)])
        pltpu.sync_copy(vals_v, vn_out.at[pl.ds(base, epw)])

        pltpu.sync_copy(pb_hbm, pb_v)
        pltpu.sync_copy(so_hbm.at[pl.ds(base, epw)], src_v)
        pltpu.sync_copy(do_hbm.at[pl.ds(base, epw)], dst_v)
        pltpu.sync_copy(adjo_hbm.at[pl.ds(base, epw)], adj_v)

        def ogrp(gi, _):
            eb = gi * _L
            s_g = src_v[pl.ds(eb, _L)]
            d_g = dst_v[pl.ds(eb, _L)]
            a_g = adj_v[pl.ds(eb, _L)]
            pbg = plsc.load_gather(
                pb_v, [jnp.right_shift(s_g, 7), jnp.bitwise_and(s_g, 127)])
            keys_v[pl.ds(eb, _L)] = s_g * n + d_g
            vals_v[pl.ds(eb, _L)] = -(a_g * pbg)
            return 0

        lax.fori_loop(0, epw // _L, ogrp, 0)
        pltpu.sync_copy(keys_v, ko_out.at[pl.ds(base, epw)])
        pltpu.sync_copy(vals_v, vo_out.at[pl.ds(base, epw)])

    return edge_vals


def _make_scatter(n, e):
    info = plsc.get_sparse_core_info()
    nc, ns = info.num_cores, info.num_subcores
    rwords = _RROWS * n
    nranges = (n * n) // rwords
    npass = nranges // nc
    sl = e // ns
    sr = sl // 128
    span = rwords // ns
    mesh = plsc.VectorSubcoreMesh(core_axis_name="c", subcore_axis_name="s")

    @functools.partial(
        pl.kernel,
        mesh=mesh,
        out_type=jax.ShapeDtypeStruct((n * n,), jnp.float32),
        scratch_types=[
            pltpu.VMEM((8, 128), jnp.int32),
            pltpu.VMEM((8, 128), jnp.float32),
            pltpu.VMEM((8, 128), jnp.int32),
            pltpu.VMEM((8, 128), jnp.float32),
            pltpu.VMEM((_NPASS, _BCAP), jnp.int32),
            pltpu.VMEM((_NPASS, _BCAP), jnp.float32),
            pltpu.VMEM((4096,), jnp.float32),
            pltpu.VMEM_SHARED((rwords,), jnp.float32),
            pltpu.SemaphoreType.DMA,
        ],
        compiler_params=pltpu.CompilerParams(use_tc_tiling_on_sc=False, needs_layout_passes=False),
    )
    def scatter(kn_hbm, vn_hbm, ko_hbm, vo_hbm, out_hbm,
                knew, vnew, kold, vold, bidx, bval, zbuf, shared, sem_s):
        cc = lax.axis_index("c")
        s = lax.axis_index("s")

        zv = jnp.zeros((_L,), jnp.float32)
        zi = jnp.zeros((_L,), jnp.int32)

        def zb(i, _):
            zbuf[pl.ds(i * _L, _L)] = zv
            return 0

        lax.fori_loop(0, 4096 // _L, zb, 0)

        for p in range(_NPASS):
            def zk(i, _, p=p):
                bidx[p, pl.ds(i * _L, _L)] = zi
                bval[p, pl.ds(i * _L, _L)] = zv
                return 0

            lax.fori_loop(0, _BCAP // _L, zk, 0)

        rshift = rwords.bit_length() - 1

        def route(kk, vv, cnts):
            rid = jnp.right_shift(kk, rshift)
            mine = jnp.bitwise_and(rid, nc - 1) == cc
            loc = jnp.bitwise_and(kk, rwords - 1)
            pp = jnp.right_shift(rid, 1)
            out = []
            for p in range(_NPASS):
                m = mine & (pp == p)
                plsc.store_compressed(bidx.at[p, pl.ds(cnts[p], _L)], loc, mask=m)
                plsc.store_compressed(bval.at[p, pl.ds(cnts[p], _L)], vv, mask=m)
                out.append(jnp.minimum(cnts[p] + jnp.sum(m.astype(jnp.int32)),
                                       _BCAP - _L))
            return tuple(out)

        def bchunk(t, cnts):
            row0 = s * sr + t * 8
            pltpu.async_copy(kn_hbm.at[pl.ds(row0, 8)], knew, sem_s)
            pltpu.async_copy(vn_hbm.at[pl.ds(row0, 8)], vnew, sem_s)
            pltpu.async_copy(ko_hbm.at[pl.ds(row0, 8)], kold, sem_s)
            pltpu.async_copy(vo_hbm.at[pl.ds(row0, 8)], vold, sem_s)
            pltpu.make_async_copy(kn_hbm.at[pl.ds(row0, 8)], knew,
                                  sem_s).wait()
            pltpu.make_async_copy(vn_hbm.at[pl.ds(row0, 8)], vnew,
                                  sem_s).wait()
            pltpu.make_async_copy(ko_hbm.at[pl.ds(row0, 8)], kold,
                                  sem_s).wait()
            pltpu.make_async_copy(vo_hbm.at[pl.ds(row0, 8)], vold,
                                  sem_s).wait()

            def bi(j, cnts):
                for g in range(8):
                    cnts = route(knew[j, pl.ds(g * _L, _L)],
                                 vnew[j, pl.ds(g * _L, _L)], cnts)
                    cnts = route(kold[j, pl.ds(g * _L, _L)],
                                 vold[j, pl.ds(g * _L, _L)], cnts)
                return cnts

            return lax.fori_loop(0, 8, bi, cnts)

        cnts = lax.fori_loop(0, sr // 8, bchunk, (jnp.int32(0),) * _NPASS)

        for p in range(_NPASS):
            rid = p * nc + cc
            lo = rid * rwords

            def zr(z, _):
                pltpu.async_copy(zbuf,
                                 shared.at[pl.ds(s * span + z * 4096, 4096)],
                                 sem_s)
                return 0

            lax.fori_loop(0, span // 4096, zr, 0)

            def zr_wait(z, _):
                pltpu.make_async_copy(
                    zbuf, shared.at[pl.ds(s * span + z * 4096, 4096)],
                    sem_s).wait()
                return 0

            lax.fori_loop(0, span // 4096, zr_wait, 0)
            plsc.subcore_barrier()

            nr = jnp.right_shift(cnts[p] + 127, 7)

            def fire(r, _, p=p):
                pltpu.async_copy(bval.at[p, pl.ds(r * 128, 128)],
                                 shared.at[bidx.at[p, pl.ds(r * 128, 128)]],
                                 sem_s, add=True)
                return 0

            lax.fori_loop(0, nr, fire, 0)

            def drain(r, _, p=p):
                pltpu.make_async_copy(
                    bval.at[p, pl.ds(r * 128, 128)],
                    shared.at[bidx.at[p, pl.ds(r * 128, 128)]],
                    sem_s).wait()
                return 0

            lax.fori_loop(0, nr, drain, 0)
            plsc.subcore_barrier()

            pltpu.sync_copy(shared.at[pl.ds(s * span, span)],
                            out_hbm.at[pl.ds(lo + s * span, span)])

    return scatter


def kernel(next_feature, adj_new_vals, feature, adj_old_vals, alpha, beta,
           gamma, persona, edge_new_idx, edge_old_idx, time):
    n, d = feature.shape
    e = edge_new_idx.shape[1]
    p = persona.shape[2]

    pt = lax.dynamic_index_in_dim(persona, time, 0, keepdims=False)
    ptp = jnp.pad(pt, ((0, 0), (0, d - p)))
    w = (jnp.zeros((d, d), jnp.float32)
         .at[:p, 0].set(alpha)
         .at[:p, 1].set(gamma)
         .at[:p, 2].set(beta))

    a, b, pb2 = _make_prep(n, d)(next_feature, feature, ptp, w)

    src = edge_new_idx[0].astype(jnp.int32)
    dst = edge_new_idx[1].astype(jnp.int32)
    so = edge_old_idx[0].astype(jnp.int32)
    do = edge_old_idx[1].astype(jnp.int32)

    kn, vn, ko, vo = _make_edge_vals(n, d, e)(
        a, b, src, dst, adj_new_vals, so, do, adj_old_vals, pb2)

    out_flat = _make_scatter(n, e)(
        kn.reshape(-1, 128), vn.reshape(-1, 128),
        ko.reshape(-1, 128), vo.reshape(-1, 128))

    return out_flat.reshape(n, n)
